# baseline (speedup 1.0000x reference)
import functools

import jax
import jax.numpy as jnp
from jax import lax
from jax.experimental import pallas as pl
from jax.experimental.pallas import tpu as pltpu
from jax.experimental.pallas import tpu_sc as plsc

V = 1000000
D = 64
B = 16384

_NC = 2
_NS = 16
_NW = _NC * _NS
_BPW = B // _NW
_L = 16
_G = 4
_NBLK = _BPW // _G
_VFULL = (V // 128) * 128

_mesh = plsc.VectorSubcoreMesh(core_axis_name="c", subcore_axis_name="s")


@functools.partial(
    pl.kernel,
    out_type=(
        jax.ShapeDtypeStruct((D, B), jnp.float32),
        jax.ShapeDtypeStruct((D, B), jnp.float32),
        jax.ShapeDtypeStruct((B,), jnp.float32),
        jax.ShapeDtypeStruct((B,), jnp.float32),
    ),
    mesh=_mesh,
    compiler_params=pltpu.CompilerParams(use_tc_tiling_on_sc=True,
                                         needs_layout_passes=False),
    scratch_types=[
        pltpu.VMEM((_BPW,), jnp.int32),
        pltpu.VMEM((_BPW,), jnp.int32),
        pltpu.VMEM((2, _G, D, 128), jnp.float32),
        pltpu.VMEM((D, _BPW), jnp.float32),
        pltpu.VMEM((4, 128), jnp.int32),
        pltpu.VMEM((128, 128), jnp.float32),
        pltpu.VMEM((4 * _L,), jnp.float32),
        pltpu.VMEM((_BPW,), jnp.float32),
        pltpu.VMEM((_BPW,), jnp.float32),
        pltpu.SemaphoreType.DMA,
        pltpu.SemaphoreType.DMA,
    ],
)
def _gather_kernel(id_i, id_j, WiT, WjT, Bi128, Bj128, Bit, Bjt,
                   wi_o, wj_o, bi_o, bj_o,
                   ii_v, ij_v, slab, w_vT,
                   brows, bchunk, btail, bi_v, bj_v,
                   sem, bsem):
    wid = lax.axis_index("s") * _NC + lax.axis_index("c")
    base = pl.multiple_of(wid * _BPW, _BPW)
    pltpu.sync_copy(id_i.at[pl.ds(base, _BPW)], ii_v)
    pltpu.sync_copy(id_j.at[pl.ds(base, _BPW)], ij_v)

    lane_iota = lax.iota(jnp.int32, _L)

    def gather_table(tab, iv, out_vT):
        nsub = _L // _G

        def fire(vvec, s, bank):
            for l in range(_G):
                vcol = pl.multiple_of(
                    lax.bitwise_and(vvec[s * _G + l], -128), 128)
                pltpu.async_copy(tab.at[:, pl.ds(vcol, 128)],
                                 slab.at[bank, l], sem)

        def drain(vvec, s, bank, iofs):
            for l in range(_G):
                pltpu.make_async_copy(tab.at[:, pl.ds(0, 128)],
                                      slab.at[bank, l], sem).wait()
                lane = lax.bitwise_and(vvec[s * _G + l], 127)
                i = iofs + s * _G + l
                blk = slab.at[bank, l]
                for dblk in range(D // _L):
                    vals = plsc.load_gather(
                        blk, [lane_iota + dblk * _L,
                              jnp.broadcast_to(lane, (_L,))])
                    plsc.store_scatter(
                        out_vT, [lane_iota + dblk * _L,
                                 jnp.broadcast_to(i, (_L,))], vals)

        v0 = iv[pl.ds(0, _L)]
        fire(v0, 0, 0)
        for s in range(1, nsub):
            fire(v0, s, s % 2)
            drain(v0, s - 1, (s - 1) % 2, 0)

        def loop_body(g, _):
            vvec = iv[pl.ds(g * _L, _L)]
            pvec = iv[pl.ds((g - 1) * _L, _L)]
            fire(vvec, 0, 0)
            drain(pvec, nsub - 1, (nsub - 1) % 2, (g - 1) * _L)
            for s in range(1, nsub):
                fire(vvec, s, s % 2)
                drain(vvec, s - 1, (s - 1) % 2, g * _L)
            return 0

        nblk = _BPW // _L
        lax.fori_loop(1, nblk, loop_body, 0)
        vlast = iv[pl.ds((nblk - 1) * _L, _L)]
        drain(vlast, nsub - 1, (nsub - 1) % 2, (nblk - 1) * _L)

    out_sl = pl.ds(base, _BPW)
    gather_table(WiT, ii_v, w_vT)
    pltpu.sync_copy(w_vT, wi_o.at[:, out_sl])
    gather_table(WjT, ij_v, w_vT)
    pltpu.sync_copy(w_vT, wj_o.at[:, out_sl])

    def gather_bias(b128, btab_tail, iv, out_b):
        pltpu.sync_copy(btab_tail, btail)
        for k in range(_BPW // 128):
            for t in range(128 // _L):
                sl = pl.ds(t * _L, _L)
                v = iv[pl.ds(k * 128 + t * _L, _L)]
                brows[k, sl] = jnp.minimum(
                    lax.shift_right_logical(v, 7), (_VFULL // 128) - 1)
        for k in range(_BPW // 128):
            pltpu.async_copy(b128.at[brows.at[k]], bchunk, bsem).wait()
            for t in range(128 // _L):
                v = iv[pl.ds(k * 128 + t * _L, _L)]
                lane = lax.bitwise_and(v, 127)
                vals = plsc.load_gather(bchunk, [lane_iota + t * _L, lane])
                tidx = jnp.clip(v - _VFULL, 0, 63)
                tvals = plsc.load_gather(btail, [tidx])
                vals = jnp.where(v >= _VFULL, tvals, vals)
                out_b[pl.ds(k * 128 + t * _L, _L)] = vals

    gather_bias(Bi128, Bit, ii_v, bi_v)
    gather_bias(Bj128, Bjt, ij_v, bj_v)

    pltpu.sync_copy(bi_v, bi_o.at[out_sl])
    pltpu.sync_copy(bj_v, bj_o.at[out_sl])


def kernel(id_i, id_j, Wi, Wj, Bi, Bj):
    WiT = Wi.T
    WjT = Wj.T
    Bi128 = Bi[:_VFULL, 0].reshape(_VFULL // 128, 128)
    Bj128 = Bj[:_VFULL, 0].reshape(_VFULL // 128, 128)
    Bit = jnp.pad(Bi[_VFULL:, 0], (0, 64 - (V - _VFULL)))
    Bjt = jnp.pad(Bj[_VFULL:, 0], (0, 64 - (V - _VFULL)))
    wiT, wjT, bi, bj = _gather_kernel(id_i, id_j, WiT, WjT,
                                      Bi128, Bj128, Bit, Bjt)
    return wiT.T, wjT.T, bi.reshape(B, 1), bj.reshape(B, 1)


# speedup vs baseline: 1.0043x; 1.0043x over previous
import functools

import jax
import jax.numpy as jnp
from jax import lax
from jax.experimental import pallas as pl
from jax.experimental.pallas import tpu as pltpu
from jax.experimental.pallas import tpu_sc as plsc

V = 1000000
D = 64
B = 16384

_NC = 2
_NS = 16
_NW = _NC * _NS
_BPW = B // _NW
_L = 16
_G = 4
_NBLK = _BPW // _G
_VFULL = (V // 128) * 128

_mesh = plsc.VectorSubcoreMesh(core_axis_name="c", subcore_axis_name="s")


@functools.partial(
    pl.kernel,
    out_type=(
        jax.ShapeDtypeStruct((D, B), jnp.float32),
        jax.ShapeDtypeStruct((D, B), jnp.float32),
        jax.ShapeDtypeStruct((B,), jnp.float32),
        jax.ShapeDtypeStruct((B,), jnp.float32),
    ),
    mesh=_mesh,
    compiler_params=pltpu.CompilerParams(use_tc_tiling_on_sc=True,
                                         needs_layout_passes=False),
    scratch_types=[
        pltpu.VMEM((_BPW,), jnp.int32),
        pltpu.VMEM((_BPW,), jnp.int32),
        pltpu.VMEM((2, _G, D, 128), jnp.float32),
        pltpu.VMEM((D, _BPW), jnp.float32),
        pltpu.VMEM((16, 64), jnp.int32),
        pltpu.VMEM((2, 64, 128), jnp.float32),
        pltpu.VMEM((4 * _L,), jnp.float32),
        pltpu.VMEM((4 * _L,), jnp.float32),
        pltpu.VMEM((_BPW,), jnp.float32),
        pltpu.VMEM((_BPW,), jnp.float32),
        pltpu.SemaphoreType.DMA,
        pltpu.SemaphoreType.DMA,
    ],
)
def _gather_kernel(id_i, id_j, WiT, WjT, Bi128, Bj128, Bit, Bjt,
                   wi_o, wj_o, bi_o, bj_o,
                   ii_v, ij_v, slab, w_vT,
                   brows, bchunk, btail_i, btail_j, bi_v, bj_v,
                   sem, bsem):
    wid = lax.axis_index("s") * _NC + lax.axis_index("c")
    base = pl.multiple_of(wid * _BPW, _BPW)
    pltpu.sync_copy(id_i.at[pl.ds(base, _BPW)], ii_v)
    pltpu.sync_copy(id_j.at[pl.ds(base, _BPW)], ij_v)

    lane_iota = lax.iota(jnp.int32, _L)

    def gather_table(tab, iv, out_vT, bias_overlap=False):
        nsub = _L // _G

        def fire(vvec, s, bank):
            for l in range(_G):
                vcol = pl.multiple_of(
                    lax.bitwise_and(vvec[s * _G + l], -128), 128)
                pltpu.async_copy(tab.at[:, pl.ds(vcol, 128)],
                                 slab.at[bank, l], sem)

        def drain(vvec, s, bank, iofs):
            for l in range(_G):
                pltpu.make_async_copy(tab.at[:, pl.ds(0, 128)],
                                      slab.at[bank, l], sem).wait()
                lane = lax.bitwise_and(vvec[s * _G + l], 127)
                i = iofs + s * _G + l
                blk = slab.at[bank, l]
                for dblk in range(D // _L):
                    vals = plsc.load_gather(
                        blk, [lane_iota + dblk * _L,
                              jnp.broadcast_to(lane, (_L,))])
                    plsc.store_scatter(
                        out_vT, [lane_iota + dblk * _L,
                                 jnp.broadcast_to(i, (_L,))], vals)

        v0 = iv[pl.ds(0, _L)]
        fire(v0, 0, 0)
        for s in range(1, nsub):
            fire(v0, s, s % 2)
            drain(v0, s - 1, (s - 1) % 2, 0)

        def loop_body(g, _):
            vvec = iv[pl.ds(g * _L, _L)]
            pvec = iv[pl.ds((g - 1) * _L, _L)]
            fire(vvec, 0, 0)
            drain(pvec, nsub - 1, (nsub - 1) % 2, (g - 1) * _L)
            for s in range(1, nsub):
                fire(vvec, s, s % 2)
                drain(vvec, s - 1, (s - 1) % 2, g * _L)
            if bias_overlap:
                for k in range(16):
                    @pl.when(g == 2 * k + 1)
                    def _(k=k):
                        bias_step(k)
            return 0

        nblk = _BPW // _L
        lax.fori_loop(1, nblk, loop_body, 0)
        vlast = iv[pl.ds((nblk - 1) * _L, _L)]
        drain(vlast, nsub - 1, (nsub - 1) % 2, (nblk - 1) * _L)

    pltpu.sync_copy(Bit, btail_i)
    pltpu.sync_copy(Bjt, btail_j)
    for k in range(16):
        iv = ii_v if k < 8 else ij_v
        for t in range(4):
            v = iv[pl.ds((k % 8) * 64 + t * _L, _L)]
            brows[k, pl.ds(t * _L, _L)] = jnp.minimum(
                lax.shift_right_logical(v, 7), (_VFULL // 128) - 1)

    def bias_fire(k):
        b128 = Bi128 if k < 8 else Bj128
        pltpu.async_copy(b128.at[brows.at[k]], bchunk.at[k % 2], bsem)

    def bias_step(k):
        b128 = Bi128 if k < 8 else Bj128
        iv = ii_v if k < 8 else ij_v
        btail = btail_i if k < 8 else btail_j
        out_b = bi_v if k < 8 else bj_v
        pltpu.make_async_copy(b128.at[brows.at[k]], bchunk.at[k % 2],
                              bsem).wait()
        blk = bchunk.at[k % 2]
        for t in range(4):
            v = iv[pl.ds((k % 8) * 64 + t * _L, _L)]
            lane = lax.bitwise_and(v, 127)
            vals = plsc.load_gather(blk, [lane_iota + t * _L, lane])
            tidx = jnp.clip(v - _VFULL, 0, 63)
            tvals = plsc.load_gather(btail, [tidx])
            vals = jnp.where(v >= _VFULL, tvals, vals)
            out_b[pl.ds((k % 8) * 64 + t * _L, _L)] = vals
        if k + 2 < 16:
            bias_fire(k + 2)

    bias_fire(0)
    bias_fire(1)

    out_sl = pl.ds(base, _BPW)
    gather_table(WiT, ii_v, w_vT, bias_overlap=True)
    pltpu.sync_copy(w_vT, wi_o.at[:, out_sl])
    gather_table(WjT, ij_v, w_vT, bias_overlap=False)
    pltpu.sync_copy(w_vT, wj_o.at[:, out_sl])

    pltpu.sync_copy(bi_v, bi_o.at[out_sl])
    pltpu.sync_copy(bj_v, bj_o.at[out_sl])


def kernel(id_i, id_j, Wi, Wj, Bi, Bj):
    WiT = Wi.T
    WjT = Wj.T
    Bi128 = Bi[:_VFULL, 0].reshape(_VFULL // 128, 128)
    Bj128 = Bj[:_VFULL, 0].reshape(_VFULL // 128, 128)
    Bit = jnp.pad(Bi[_VFULL:, 0], (0, 64 - (V - _VFULL)))
    Bjt = jnp.pad(Bj[_VFULL:, 0], (0, 64 - (V - _VFULL)))
    wiT, wjT, bi, bj = _gather_kernel(id_i, id_j, WiT, WjT,
                                      Bi128, Bj128, Bit, Bjt)
    return wiT.T, wjT.T, bi.reshape(B, 1), bj.reshape(B, 1)
